# baseline (speedup 1.0000x reference)
import jax
import jax.numpy as jnp
from jax import lax
from jax.experimental import pallas as pl
from jax.experimental.pallas import tpu as pltpu
from jax.experimental.pallas import tpu_sc as plsc

N = 10000
E = 320000
HIDDEN = 128
TRANS = 16
MASK_DIM = 32
K = 3

NUM_CORES = 2
NUM_SUBCORES = 16
LANES = 16

N_PAD = 10240
SLICE = N_PAD // NUM_SUBCORES
EPT = E // NUM_SUBCORES
GCH = 80
NCH = EPT // GCH
NB = 10
BE = 16000
PACK = 8


def _mesh():
    return plsc.VectorSubcoreMesh(
        core_axis_name="c", subcore_axis_name="s",
        num_cores=NUM_CORES, num_subcores=NUM_SUBCORES)


_SC_PARAMS = pltpu.CompilerParams(
    use_tc_tiling_on_sc=False, needs_layout_passes=False)


def _proj_body(x_ref, ws_ref, wd_ref, bea_ref, ps_ref, pd_ref):
    x = x_ref[...]
    ps_ref[...] = jnp.dot(x, ws_ref[...], preferred_element_type=jnp.float32)
    pd_ref[...] = (jnp.dot(x, wd_ref[...], preferred_element_type=jnp.float32)
                   + bea_ref[...])


def _node_proj(x_pad, w_s, w_d, b_ea2):
    return pl.pallas_call(
        _proj_body,
        out_shape=[jax.ShapeDtypeStruct((N_PAD, TRANS), jnp.float32),
                   jax.ShapeDtypeStruct((N_PAD, TRANS), jnp.float32)],
    )(x_pad, w_s, w_d, b_ea2)


def _edge_gather_body(ps_hbm, pd_hbm, sp_hbm, dm_hbm, g_hbm,
                      sidx, didx, bufs, gsem, wsem, sh_ps, sh_pd):
    c = lax.axis_index("c")
    t = lax.axis_index("s")
    ebase = t * EPT
    nrow = N_PAD // NUM_SUBCORES
    pltpu.sync_copy(ps_hbm.at[pl.ds(t * nrow, nrow)],
                    sh_ps.at[pl.ds(t * nrow, nrow)])
    pltpu.sync_copy(pd_hbm.at[pl.ds(t * nrow, nrow)],
                    sh_pd.at[pl.ds(t * nrow, nrow)])

    @pl.when(c == 0)
    def _():
        pltpu.sync_copy(sp_hbm.at[pl.ds(ebase, EPT)], sidx)
        pltpu.sync_copy(sp_hbm.at[pl.ds(E + ebase, EPT)], didx)

    @pl.when(c == 1)
    def _():
        pltpu.sync_copy(dm_hbm.at[pl.ds(ebase, EPT)], sidx)
        pltpu.sync_copy(dm_hbm.at[pl.ds(E + ebase, EPT)], didx)

    plsc.subcore_barrier()
    bs, bd, ob = bufs
    grow = GCH * TRANS // 128

    def add_rows(rs, rd, ro):
        def body(i, _):
            ro[i // PACK, pl.ds((i % PACK) * TRANS, TRANS)] = rs[i] + rd[i]
            return 0
        lax.fori_loop(0, GCH, body, 0)

    def group(gi, _):
        j0 = gi * NB
        for b in range(NB):
            @pl.when(gi > 0)
            def _():
                pltpu.make_async_copy(
                    ob[b], g_hbm.at[c, pl.ds(0, grow)], wsem[b]).wait()
            o = (j0 + b) * GCH
            pltpu.async_copy(sh_ps.at[sidx.at[pl.ds(o, GCH)]], bs[b], gsem[b])
            pltpu.async_copy(sh_pd.at[didx.at[pl.ds(o, GCH)]], bd[b], gsem[b])
        for b in range(NB):
            o = (j0 + b) * GCH
            pltpu.make_async_copy(
                sh_ps.at[sidx.at[pl.ds(o, GCH)]], bs[b], gsem[b]).wait()
            pltpu.make_async_copy(
                sh_pd.at[didx.at[pl.ds(o, GCH)]], bd[b], gsem[b]).wait()
            add_rows(bs[b], bd[b], ob[b])
            pltpu.async_copy(
                ob[b], g_hbm.at[c, pl.ds((ebase + o) // PACK, grow)], wsem[b])
        return 0

    lax.fori_loop(0, NCH // NB, group, 0)
    for b in range(NB):
        pltpu.make_async_copy(
            ob[b], g_hbm.at[c, pl.ds(0, grow)], wsem[b]).wait()


def _edge_gather(ps, pd, sp1, dm1):
    kfn = pl.kernel(
        _edge_gather_body,
        out_type=jax.ShapeDtypeStruct((NUM_CORES, E * TRANS // 128, 128),
                                      jnp.float32),
        mesh=_mesh(),
        scratch_types=[
            pltpu.VMEM((EPT,), jnp.int32),
            pltpu.VMEM((EPT,), jnp.int32),
            ([pltpu.VMEM((GCH, TRANS), jnp.float32) for _ in range(NB)],
             [pltpu.VMEM((GCH, TRANS), jnp.float32) for _ in range(NB)],
             [pltpu.VMEM((GCH * TRANS // 128, 128), jnp.float32)
              for _ in range(NB)]),
            [pltpu.SemaphoreType.DMA for _ in range(NB)],
            [pltpu.SemaphoreType.DMA for _ in range(NB)],
            pltpu.VMEM_SHARED((N_PAD, TRANS), jnp.float32),
            pltpu.VMEM_SHARED((N_PAD, TRANS), jnp.float32),
        ],
        compiler_params=_SC_PARAMS,
    )
    return kfn(ps, pd, sp1, dm1)


def _mlp_body(g_ref, asp_ref, adm_ref, w1t_ref, w1asp_ref, w1adm_ref,
              b1_ref, w2_ref, b2_ref, o_ref):
    for gi in range(NUM_CORES):
        trans = jnp.tanh(g_ref[gi])
        a_ref = asp_ref if gi == 0 else adm_ref
        w1a_ref = w1asp_ref if gi == 0 else w1adm_ref
        h = (jnp.dot(trans, w1t_ref[gi],
                     preferred_element_type=jnp.float32)
             + jnp.dot(a_ref[...], w1a_ref[...],
                       preferred_element_type=jnp.float32)
             + b1_ref[gi])
        h = jnp.maximum(h, 0.0)
        ew = jnp.dot(h, w2_ref[gi], preferred_element_type=jnp.float32)
        o_ref[gi] = jax.nn.sigmoid(ew + b2_ref[gi])


def _edge_mlp(g2, asp2, adm2, w1t_big, w1asp_big, w1adm_big,
              b1_big, w2_big, b2_big):
    grid = (E // BE,)
    bp = BE // PACK
    full = lambda a: pl.BlockSpec(a.shape, lambda bi: (0,) * a.ndim)
    return pl.pallas_call(
        _mlp_body,
        grid=grid,
        in_specs=[
            pl.BlockSpec((NUM_CORES, bp, PACK * TRANS), lambda bi: (0, bi, 0)),
            pl.BlockSpec((bp, PACK * 4), lambda bi: (bi, 0)),
            pl.BlockSpec((bp, PACK * 1), lambda bi: (bi, 0)),
            full(w1t_big),
            full(w1asp_big),
            full(w1adm_big),
            full(b1_big),
            full(w2_big),
            full(b2_big),
        ],
        out_specs=pl.BlockSpec((NUM_CORES, bp, PACK), lambda bi: (0, bi, 0)),
        out_shape=jax.ShapeDtypeStruct((NUM_CORES, E // PACK, PACK),
                                       jnp.float32),
    )(g2, asp2, adm2, w1t_big, w1asp_big, w1adm_big, b1_big, w2_big, b2_big)


def _prop_body(sp_hbm, dm_hbm, ew_hbm, m0_hbm, out_hbm,
               src_t, dst_t, ew_t, m_cur, acc, kk_st, vv_st,
               shared_s, shared_m, rsem):
        c = lax.axis_index("c")
        t = lax.axis_index("s")
        ebase = t * EPT
        sbase = t * SLICE

        @pl.when(c == 0)
        def _():
            pltpu.sync_copy(sp_hbm.at[pl.ds(ebase, EPT)], src_t)
            pltpu.sync_copy(sp_hbm.at[pl.ds(E + ebase, EPT)], dst_t)

        @pl.when(c == 1)
        def _():
            pltpu.sync_copy(dm_hbm.at[pl.ds(ebase, EPT)], src_t)
            pltpu.sync_copy(dm_hbm.at[pl.ds(E + ebase, EPT)], dst_t)

        pltpu.sync_copy(ew_hbm.at[c, pl.ds(ebase // PACK, EPT // PACK)], ew_t)
        pltpu.sync_copy(m0_hbm, m_cur)
        pltpu.sync_copy(m0_hbm, acc)
        lane = lax.iota(jnp.int32, LANES)
        qoff = lane // PACK
        jidx = lane % PACK

        HALF = EPT // 2
        HCH = HALF // LANES
        UNROLL = 5

        def stage_chunk(j):
            off = j * LANES
            si = src_t[pl.ds(off, LANES)]
            di = dst_t[pl.ds(off, LANES)]
            w = plsc.load_gather(ew_t, [2 * j + qoff, jidx])
            ms = plsc.load_gather(m_cur, [si])
            kk, vv = plsc.sort_key_val(di, ms * w)
            for sh in (1, 2, 4, 8):
                idx2 = jnp.maximum(lane - sh, 0)
                kp = kk.at[idx2].get(mode="promise_in_bounds")
                vp = vv.at[idx2].get(mode="promise_in_bounds")
                hit = (lane >= sh) & (kp == kk)
                vv = jnp.where(hit, jnp.maximum(vv, vp), vv)
            nxt = jnp.minimum(lane + 1, LANES - 1)
            kn = kk.at[nxt].get(mode="promise_in_bounds")
            is_last = (lane == LANES - 1) | (kn != kk)
            return kk, jnp.where(is_last, vv, -1.0)

        def apply_chunk(soff):
            kk = kk_st[pl.ds(soff, LANES)]
            vv = vv_st[pl.ds(soff, LANES)]
            msk = vv >= 0.0
            cur = plsc.load_gather(acc, [kk], mask=msk)
            plsc.store_scatter(acc, [kk], jnp.maximum(vv, cur), mask=msk)

        for k in range(K):
            for h in range(2):
                hbase = h * HCH

                def p1_body(ju, _):
                    for u in range(UNROLL):
                        jj = ju * UNROLL + u
                        kk, vv = stage_chunk(hbase + jj)
                        kk_st[pl.ds(jj * LANES, LANES)] = kk
                        vv_st[pl.ds(jj * LANES, LANES)] = vv
                    return 0

                def p2_body(ju, _):
                    for u in range(UNROLL):
                        apply_chunk((ju * UNROLL + u) * LANES)
                    return 0

                lax.fori_loop(0, HCH // UNROLL, p1_body, 0)
                lax.fori_loop(0, HCH // UNROLL, p2_body, 0)
            pltpu.sync_copy(acc, shared_s.at[t])
            plsc.subcore_barrier()
            stage_in = [
                pltpu.async_copy(shared_s.at[jj, pl.ds(sbase, SLICE)],
                                 m_cur.at[pl.ds(jj * SLICE, SLICE)], rsem)
                for jj in range(NUM_SUBCORES)]
            for d in stage_in:
                d.wait()

            def red_body(r, _):
                o = r * LANES
                v = m_cur[pl.ds(o, LANES)]
                for jj in range(1, NUM_SUBCORES):
                    v = jnp.maximum(v, m_cur[pl.ds(jj * SLICE + o, LANES)])
                acc[pl.ds(o, LANES)] = v
                return 0

            lax.fori_loop(0, SLICE // LANES, red_body, 0)
            if k + 1 < K:
                pltpu.sync_copy(acc.at[pl.ds(0, SLICE)],
                                shared_m.at[pl.ds(sbase, SLICE)])
                plsc.subcore_barrier()
                pltpu.sync_copy(shared_m, m_cur)
                pltpu.sync_copy(shared_m, acc)
            else:
                pltpu.sync_copy(acc.at[pl.ds(0, SLICE)],
                                out_hbm.at[c, pl.ds(sbase, SLICE)])


def _propagate(sp1, dm1, ew3, m0):
    kfn = pl.kernel(
        _prop_body,
        out_type=jax.ShapeDtypeStruct((NUM_CORES, N_PAD), jnp.float32),
        mesh=_mesh(),
        scratch_types=[
            pltpu.VMEM((EPT,), jnp.int32),
            pltpu.VMEM((EPT,), jnp.int32),
            pltpu.VMEM((EPT // PACK, PACK), jnp.float32),
            pltpu.VMEM((N_PAD,), jnp.float32),
            pltpu.VMEM((N_PAD,), jnp.float32),
            pltpu.VMEM((EPT // 2,), jnp.int32),
            pltpu.VMEM((EPT // 2,), jnp.float32),
            pltpu.VMEM_SHARED((NUM_SUBCORES, N_PAD), jnp.float32),
            pltpu.VMEM_SHARED((N_PAD,), jnp.float32),
            pltpu.SemaphoreType.DMA,
        ],
        compiler_params=_SC_PARAMS,
    )
    return kfn(sp1, dm1, ew3, m0)


def _combine_body(prop_ref, mask_ref, o_ref):
    o_ref[...] = jnp.maximum(
        jnp.maximum(prop_ref[0:1, :], prop_ref[1:2, :]), mask_ref[...])


def _combine(prop, mask_pad2):
    return pl.pallas_call(
        _combine_body,
        out_shape=jax.ShapeDtypeStruct((1, N_PAD), jnp.float32),
    )(prop, mask_pad2)


def kernel(x, spatial_edge_index, dom_edge_index, spatial_edge_attr,
           dom_edge_attr, mask, W_ea, b_ea, W_d1, b_d1, W_d2, b_d2,
           W_p1, b_p1, W_p2, b_p2):
    f32 = jnp.float32

    x_pad = jnp.pad(x.astype(f32), ((0, N_PAD - N), (0, 0)))
    w_s = W_ea[:HIDDEN]
    w_d = W_ea[HIDDEN:]
    b_ea2 = b_ea.reshape(1, TRANS)

    sp1 = spatial_edge_index.astype(jnp.int32).reshape(2 * E)
    dm1 = dom_edge_index.astype(jnp.int32).reshape(2 * E)

    asp2 = (spatial_edge_attr.astype(f32).T.reshape(4, E // PACK, PACK)
            .transpose(1, 2, 0).reshape(E // PACK, PACK * 4))
    adm2 = (dom_edge_attr.astype(f32).T.reshape(1, E // PACK, PACK)
            .transpose(1, 2, 0).reshape(E // PACK, PACK * 1))

    w1t = jnp.stack([W_p1[4:], W_d1[1:]])
    b1 = jnp.stack([b_p1, b_d1])
    w2 = jnp.stack([W_p2, W_d2])
    b2 = jnp.stack([b_p2, b_d2])
    eye8 = jnp.eye(PACK, dtype=f32)

    def blkdiag3(w):
        return (eye8[None, :, None, :, None]
                * w[:, None, :, None, :]
                ).reshape(w.shape[0], PACK * w.shape[1], PACK * w.shape[2])

    w1t_big = blkdiag3(w1t)
    w1asp_big = blkdiag3(W_p1[None, :4])[0]
    w1adm_big = blkdiag3(W_d1[None, :1])[0]
    b1_big = jnp.tile(b1, (1, PACK))
    w2_big = blkdiag3(w2)
    b2_big = jnp.tile(b2, (1, PACK))

    m0 = jnp.pad(mask[:, 0].astype(f32), (0, N_PAD - N))
    mask_pad2 = m0.reshape(1, N_PAD)

    ps, pd = _node_proj(x_pad, w_s, w_d, b_ea2)
    g2 = _edge_gather(ps, pd, sp1, dm1)
    ew3 = _edge_mlp(g2, asp2, adm2, w1t_big, w1asp_big, w1adm_big,
                    b1_big, w2_big, b2_big)
    prop = _propagate(sp1, dm1, ew3, m0)
    out = _combine(prop, mask_pad2)
    return out[0, :N].reshape(N, 1)


# speedup vs baseline: 1.0900x; 1.0900x over previous
import jax
import jax.numpy as jnp
from jax import lax
from jax.experimental import pallas as pl
from jax.experimental.pallas import tpu as pltpu
from jax.experimental.pallas import tpu_sc as plsc

N = 10000
E = 320000
HIDDEN = 128
TRANS = 16
MASK_DIM = 32
K = 3

NUM_CORES = 2
NUM_SUBCORES = 16
LANES = 16

N_PAD = 10240
SLICE = N_PAD // NUM_SUBCORES
EPT = E // NUM_SUBCORES
GCH = 80
NCH = EPT // GCH
NB = 10
BE = 16000
PACK = 8


def _mesh():
    return plsc.VectorSubcoreMesh(
        core_axis_name="c", subcore_axis_name="s",
        num_cores=NUM_CORES, num_subcores=NUM_SUBCORES)


_SC_PARAMS = pltpu.CompilerParams(
    use_tc_tiling_on_sc=False, needs_layout_passes=False)


def _proj_body(x_ref, ws_ref, wd_ref, bea_ref, ps_ref, pd_ref):
    x = x_ref[...]
    ps_ref[...] = jnp.dot(x, ws_ref[...], preferred_element_type=jnp.float32)
    pd_ref[...] = (jnp.dot(x, wd_ref[...], preferred_element_type=jnp.float32)
                   + bea_ref[...])


def _node_proj(x_pad, w_s, w_d, b_ea2):
    return pl.pallas_call(
        _proj_body,
        out_shape=[jax.ShapeDtypeStruct((N_PAD, TRANS), jnp.float32),
                   jax.ShapeDtypeStruct((N_PAD, TRANS), jnp.float32)],
    )(x_pad, w_s, w_d, b_ea2)


def _edge_gather_body(ps_hbm, pd_hbm, sp_hbm, dm_hbm, g_hbm,
                      sidx, didx, bufs, gsem, wsem, sh_ps, sh_pd):
    c = lax.axis_index("c")
    t = lax.axis_index("s")
    ebase = t * EPT
    nrow = N_PAD // NUM_SUBCORES
    pltpu.sync_copy(ps_hbm.at[pl.ds(t * nrow, nrow)],
                    sh_ps.at[pl.ds(t * nrow, nrow)])
    pltpu.sync_copy(pd_hbm.at[pl.ds(t * nrow, nrow)],
                    sh_pd.at[pl.ds(t * nrow, nrow)])

    @pl.when(c == 0)
    def _():
        pltpu.sync_copy(sp_hbm.at[pl.ds(ebase, EPT)], sidx)
        pltpu.sync_copy(sp_hbm.at[pl.ds(E + ebase, EPT)], didx)

    @pl.when(c == 1)
    def _():
        pltpu.sync_copy(dm_hbm.at[pl.ds(ebase, EPT)], sidx)
        pltpu.sync_copy(dm_hbm.at[pl.ds(E + ebase, EPT)], didx)

    plsc.subcore_barrier()
    bs, bd, ob = bufs
    grow = GCH * TRANS // 128

    def add_rows(rs, rd, ro):
        def body(i, _):
            ro[i // PACK, pl.ds((i % PACK) * TRANS, TRANS)] = rs[i] + rd[i]
            return 0
        lax.fori_loop(0, GCH, body, 0)

    def group(gi, _):
        j0 = gi * NB
        for b in range(NB):
            @pl.when(gi > 0)
            def _():
                pltpu.make_async_copy(
                    ob[b], g_hbm.at[c, pl.ds(0, grow)], wsem[b]).wait()
            o = (j0 + b) * GCH
            pltpu.async_copy(sh_ps.at[sidx.at[pl.ds(o, GCH)]], bs[b], gsem[b])
            pltpu.async_copy(sh_pd.at[didx.at[pl.ds(o, GCH)]], bd[b], gsem[b])
        for b in range(NB):
            o = (j0 + b) * GCH
            pltpu.make_async_copy(
                sh_ps.at[sidx.at[pl.ds(o, GCH)]], bs[b], gsem[b]).wait()
            pltpu.make_async_copy(
                sh_pd.at[didx.at[pl.ds(o, GCH)]], bd[b], gsem[b]).wait()
            add_rows(bs[b], bd[b], ob[b])
            pltpu.async_copy(
                ob[b], g_hbm.at[c, pl.ds((ebase + o) // PACK, grow)], wsem[b])
        return 0

    lax.fori_loop(0, NCH // NB, group, 0)
    for b in range(NB):
        pltpu.make_async_copy(
            ob[b], g_hbm.at[c, pl.ds(0, grow)], wsem[b]).wait()


def _edge_gather(ps, pd, sp1, dm1):
    kfn = pl.kernel(
        _edge_gather_body,
        out_type=jax.ShapeDtypeStruct((NUM_CORES, E * TRANS // 128, 128),
                                      jnp.float32),
        mesh=_mesh(),
        scratch_types=[
            pltpu.VMEM((EPT,), jnp.int32),
            pltpu.VMEM((EPT,), jnp.int32),
            ([pltpu.VMEM((GCH, TRANS), jnp.float32) for _ in range(NB)],
             [pltpu.VMEM((GCH, TRANS), jnp.float32) for _ in range(NB)],
             [pltpu.VMEM((GCH * TRANS // 128, 128), jnp.float32)
              for _ in range(NB)]),
            [pltpu.SemaphoreType.DMA for _ in range(NB)],
            [pltpu.SemaphoreType.DMA for _ in range(NB)],
            pltpu.VMEM_SHARED((N_PAD, TRANS), jnp.float32),
            pltpu.VMEM_SHARED((N_PAD, TRANS), jnp.float32),
        ],
        compiler_params=_SC_PARAMS,
    )
    return kfn(ps, pd, sp1, dm1)


def _mlp_body(g_ref, asp_ref, adm_ref, w1t_ref, w1asp_ref, w1adm_ref,
              b1_ref, w2_ref, b2_ref, o_ref):
    for gi in range(NUM_CORES):
        trans = jnp.tanh(g_ref[gi])
        a_ref = asp_ref if gi == 0 else adm_ref
        w1a_ref = w1asp_ref if gi == 0 else w1adm_ref
        h = (jnp.dot(trans, w1t_ref[gi],
                     preferred_element_type=jnp.float32)
             + jnp.dot(a_ref[...], w1a_ref[...],
                       preferred_element_type=jnp.float32)
             + b1_ref[gi])
        h = jnp.maximum(h, 0.0)
        ew = jnp.dot(h, w2_ref[gi], preferred_element_type=jnp.float32)
        o_ref[gi] = jax.nn.sigmoid(ew + b2_ref[gi])


def _edge_mlp(g2, asp2, adm2, w1t_big, w1asp_big, w1adm_big,
              b1_big, w2_big, b2_big):
    grid = (E // BE,)
    bp = BE // PACK
    full = lambda a: pl.BlockSpec(a.shape, lambda bi: (0,) * a.ndim)
    return pl.pallas_call(
        _mlp_body,
        grid=grid,
        in_specs=[
            pl.BlockSpec((NUM_CORES, bp, PACK * TRANS), lambda bi: (0, bi, 0)),
            pl.BlockSpec((bp, PACK * 4), lambda bi: (bi, 0)),
            pl.BlockSpec((bp, PACK * 1), lambda bi: (bi, 0)),
            full(w1t_big),
            full(w1asp_big),
            full(w1adm_big),
            full(b1_big),
            full(w2_big),
            full(b2_big),
        ],
        out_specs=pl.BlockSpec((NUM_CORES, bp, PACK), lambda bi: (0, bi, 0)),
        out_shape=jax.ShapeDtypeStruct((NUM_CORES, E // PACK, PACK),
                                       jnp.float32),
    )(g2, asp2, adm2, w1t_big, w1asp_big, w1adm_big, b1_big, w2_big, b2_big)


def _prop_body(sp_hbm, dm_hbm, ew_hbm, m0_hbm, out_hbm,
               src_t, dst_t, ew_t, m_cur, acc,
               shared_s, shared_m, rsem):
        c = lax.axis_index("c")
        t = lax.axis_index("s")
        ebase = t * EPT
        sbase = t * SLICE

        @pl.when(c == 0)
        def _():
            pltpu.sync_copy(sp_hbm.at[pl.ds(ebase, EPT)], src_t)
            pltpu.sync_copy(sp_hbm.at[pl.ds(E + ebase, EPT)], dst_t)

        @pl.when(c == 1)
        def _():
            pltpu.sync_copy(dm_hbm.at[pl.ds(ebase, EPT)], src_t)
            pltpu.sync_copy(dm_hbm.at[pl.ds(E + ebase, EPT)], dst_t)

        pltpu.sync_copy(ew_hbm.at[c, pl.ds(ebase // PACK, EPT // PACK)], ew_t)
        pltpu.sync_copy(m0_hbm, m_cur)
        pltpu.sync_copy(m0_hbm, acc)
        lane = lax.iota(jnp.int32, LANES)
        qoff = lane // PACK
        jidx = lane % PACK

        def scatter_chunk(j):
            off = j * LANES
            si = src_t[pl.ds(off, LANES)]
            di = dst_t[pl.ds(off, LANES)]
            w = plsc.load_gather(ew_t, [2 * j + qoff, jidx])
            ms = plsc.load_gather(m_cur, [si])
            kk, vv = plsc.sort_key_val(di, ms * w)
            for sh in (1, 2, 4, 8):
                idx2 = jnp.maximum(lane - sh, 0)
                kp = kk.at[idx2].get(mode="promise_in_bounds")
                vp = vv.at[idx2].get(mode="promise_in_bounds")
                hit = (lane >= sh) & (kp == kk)
                vv = jnp.where(hit, jnp.maximum(vv, vp), vv)
            nxt = jnp.minimum(lane + 1, LANES - 1)
            kn = kk.at[nxt].get(mode="promise_in_bounds")
            is_last = (lane == LANES - 1) | (kn != kk)
            cur = plsc.load_gather(acc, [kk], mask=is_last)
            plsc.store_scatter(acc, [kk], jnp.maximum(vv, cur), mask=is_last)

        UNROLL = 4

        def edge_body(ju, _):
            for u in range(UNROLL):
                scatter_chunk(ju * UNROLL + u)
            return 0

        for k in range(K):
            lax.fori_loop(0, EPT // LANES // UNROLL, edge_body, 0)
            pltpu.sync_copy(acc, shared_s.at[t])
            plsc.subcore_barrier()
            stage_in = [
                pltpu.async_copy(shared_s.at[jj, pl.ds(sbase, SLICE)],
                                 m_cur.at[pl.ds(jj * SLICE, SLICE)], rsem)
                for jj in range(NUM_SUBCORES)]
            for d in stage_in:
                d.wait()

            def red_body(r, _):
                o = r * LANES
                v = m_cur[pl.ds(o, LANES)]
                for jj in range(1, NUM_SUBCORES):
                    v = jnp.maximum(v, m_cur[pl.ds(jj * SLICE + o, LANES)])
                acc[pl.ds(o, LANES)] = v
                return 0

            lax.fori_loop(0, SLICE // LANES, red_body, 0)
            if k + 1 < K:
                pltpu.sync_copy(acc.at[pl.ds(0, SLICE)],
                                shared_m.at[pl.ds(sbase, SLICE)])
                plsc.subcore_barrier()
                pltpu.sync_copy(shared_m, m_cur)
                pltpu.sync_copy(shared_m, acc)
            else:
                pltpu.sync_copy(acc.at[pl.ds(0, SLICE)],
                                out_hbm.at[c, pl.ds(sbase, SLICE)])


def _propagate(sp1, dm1, ew3, m0):
    kfn = pl.kernel(
        _prop_body,
        out_type=jax.ShapeDtypeStruct((NUM_CORES, N_PAD), jnp.float32),
        mesh=_mesh(),
        scratch_types=[
            pltpu.VMEM((EPT,), jnp.int32),
            pltpu.VMEM((EPT,), jnp.int32),
            pltpu.VMEM((EPT // PACK, PACK), jnp.float32),
            pltpu.VMEM((N_PAD,), jnp.float32),
            pltpu.VMEM((N_PAD,), jnp.float32),
            pltpu.VMEM_SHARED((NUM_SUBCORES, N_PAD), jnp.float32),
            pltpu.VMEM_SHARED((N_PAD,), jnp.float32),
            pltpu.SemaphoreType.DMA,
        ],
        compiler_params=_SC_PARAMS,
    )
    return kfn(sp1, dm1, ew3, m0)


def _combine_body(prop_ref, mask_ref, o_ref):
    o_ref[...] = jnp.maximum(
        jnp.maximum(prop_ref[0:1, :], prop_ref[1:2, :]), mask_ref[...])


def _combine(prop, mask_pad2):
    return pl.pallas_call(
        _combine_body,
        out_shape=jax.ShapeDtypeStruct((1, N_PAD), jnp.float32),
    )(prop, mask_pad2)


def kernel(x, spatial_edge_index, dom_edge_index, spatial_edge_attr,
           dom_edge_attr, mask, W_ea, b_ea, W_d1, b_d1, W_d2, b_d2,
           W_p1, b_p1, W_p2, b_p2):
    f32 = jnp.float32

    x_pad = jnp.pad(x.astype(f32), ((0, N_PAD - N), (0, 0)))
    w_s = W_ea[:HIDDEN]
    w_d = W_ea[HIDDEN:]
    b_ea2 = b_ea.reshape(1, TRANS)

    sp1 = spatial_edge_index.astype(jnp.int32).reshape(2 * E)
    dm1 = dom_edge_index.astype(jnp.int32).reshape(2 * E)

    asp2 = (spatial_edge_attr.astype(f32).T.reshape(4, E // PACK, PACK)
            .transpose(1, 2, 0).reshape(E // PACK, PACK * 4))
    adm2 = (dom_edge_attr.astype(f32).T.reshape(1, E // PACK, PACK)
            .transpose(1, 2, 0).reshape(E // PACK, PACK * 1))

    w1t = jnp.stack([W_p1[4:], W_d1[1:]])
    b1 = jnp.stack([b_p1, b_d1])
    w2 = jnp.stack([W_p2, W_d2])
    b2 = jnp.stack([b_p2, b_d2])
    eye8 = jnp.eye(PACK, dtype=f32)

    def blkdiag3(w):
        return (eye8[None, :, None, :, None]
                * w[:, None, :, None, :]
                ).reshape(w.shape[0], PACK * w.shape[1], PACK * w.shape[2])

    w1t_big = blkdiag3(w1t)
    w1asp_big = blkdiag3(W_p1[None, :4])[0]
    w1adm_big = blkdiag3(W_d1[None, :1])[0]
    b1_big = jnp.tile(b1, (1, PACK))
    w2_big = blkdiag3(w2)
    b2_big = jnp.tile(b2, (1, PACK))

    m0 = jnp.pad(mask[:, 0].astype(f32), (0, N_PAD - N))
    mask_pad2 = m0.reshape(1, N_PAD)

    ps, pd = _node_proj(x_pad, w_s, w_d, b_ea2)
    g2 = _edge_gather(ps, pd, sp1, dm1)
    ew3 = _edge_mlp(g2, asp2, adm2, w1t_big, w1asp_big, w1adm_big,
                    b1_big, w2_big, b2_big)
    prop = _propagate(sp1, dm1, ew3, m0)
    out = _combine(prop, mask_pad2)
    return out[0, :N].reshape(N, 1)


# speedup vs baseline: 1.0939x; 1.0036x over previous
import jax
import jax.numpy as jnp
from jax import lax
from jax.experimental import pallas as pl
from jax.experimental.pallas import tpu as pltpu
from jax.experimental.pallas import tpu_sc as plsc

N = 10000
E = 320000
HIDDEN = 128
TRANS = 16
MASK_DIM = 32
K = 3

NUM_CORES = 2
NUM_SUBCORES = 16
LANES = 16

N_PAD = 10240
SLICE = N_PAD // NUM_SUBCORES
EPT = E // NUM_SUBCORES
GCH = 80
NCH = EPT // GCH
NB = 10
BE = 16000
PACK = 8


def _mesh():
    return plsc.VectorSubcoreMesh(
        core_axis_name="c", subcore_axis_name="s",
        num_cores=NUM_CORES, num_subcores=NUM_SUBCORES)


_SC_PARAMS = pltpu.CompilerParams(
    use_tc_tiling_on_sc=False, needs_layout_passes=False)


def _proj_body(x_ref, ws_ref, wd_ref, bea_ref, ps_ref, pd_ref):
    x = x_ref[...]
    ps_ref[...] = jnp.dot(x, ws_ref[...], preferred_element_type=jnp.float32)
    pd_ref[...] = (jnp.dot(x, wd_ref[...], preferred_element_type=jnp.float32)
                   + bea_ref[...])


def _node_proj(x_pad, w_s, w_d, b_ea2):
    return pl.pallas_call(
        _proj_body,
        out_shape=[jax.ShapeDtypeStruct((N_PAD, TRANS), jnp.float32),
                   jax.ShapeDtypeStruct((N_PAD, TRANS), jnp.float32)],
    )(x_pad, w_s, w_d, b_ea2)


def _edge_gather_body(ps_hbm, pd_hbm, sp_hbm, dm_hbm, g_hbm,
                      sidx, didx, bufs, gsem, wsem, sh_ps, sh_pd):
    c = lax.axis_index("c")
    t = lax.axis_index("s")
    ebase = t * EPT
    nrow = N_PAD // NUM_SUBCORES
    pltpu.sync_copy(ps_hbm.at[pl.ds(t * nrow, nrow)],
                    sh_ps.at[pl.ds(t * nrow, nrow)])
    pltpu.sync_copy(pd_hbm.at[pl.ds(t * nrow, nrow)],
                    sh_pd.at[pl.ds(t * nrow, nrow)])

    @pl.when(c == 0)
    def _():
        pltpu.sync_copy(sp_hbm.at[pl.ds(ebase, EPT)], sidx)
        pltpu.sync_copy(sp_hbm.at[pl.ds(E + ebase, EPT)], didx)

    @pl.when(c == 1)
    def _():
        pltpu.sync_copy(dm_hbm.at[pl.ds(ebase, EPT)], sidx)
        pltpu.sync_copy(dm_hbm.at[pl.ds(E + ebase, EPT)], didx)

    plsc.subcore_barrier()
    bs, bd, ob = bufs
    grow = GCH * TRANS // 128

    def add_rows(rs, rd, ro):
        def body(i, _):
            ro[i // PACK, pl.ds((i % PACK) * TRANS, TRANS)] = rs[i] + rd[i]
            return 0
        lax.fori_loop(0, GCH, body, 0)

    def group(gi, _):
        j0 = gi * NB
        for b in range(NB):
            @pl.when(gi > 0)
            def _():
                pltpu.make_async_copy(
                    ob[b], g_hbm.at[c, pl.ds(0, grow)], wsem[b]).wait()
            o = (j0 + b) * GCH
            pltpu.async_copy(sh_ps.at[sidx.at[pl.ds(o, GCH)]], bs[b], gsem[b])
            pltpu.async_copy(sh_pd.at[didx.at[pl.ds(o, GCH)]], bd[b], gsem[b])
        for b in range(NB):
            o = (j0 + b) * GCH
            pltpu.make_async_copy(
                sh_ps.at[sidx.at[pl.ds(o, GCH)]], bs[b], gsem[b]).wait()
            pltpu.make_async_copy(
                sh_pd.at[didx.at[pl.ds(o, GCH)]], bd[b], gsem[b]).wait()
            add_rows(bs[b], bd[b], ob[b])
            pltpu.async_copy(
                ob[b], g_hbm.at[c, pl.ds((ebase + o) // PACK, grow)], wsem[b])
        return 0

    lax.fori_loop(0, NCH // NB, group, 0)
    for b in range(NB):
        pltpu.make_async_copy(
            ob[b], g_hbm.at[c, pl.ds(0, grow)], wsem[b]).wait()


def _edge_gather(ps, pd, sp1, dm1):
    kfn = pl.kernel(
        _edge_gather_body,
        out_type=jax.ShapeDtypeStruct((NUM_CORES, E * TRANS // 128, 128),
                                      jnp.float32),
        mesh=_mesh(),
        scratch_types=[
            pltpu.VMEM((EPT,), jnp.int32),
            pltpu.VMEM((EPT,), jnp.int32),
            ([pltpu.VMEM((GCH, TRANS), jnp.float32) for _ in range(NB)],
             [pltpu.VMEM((GCH, TRANS), jnp.float32) for _ in range(NB)],
             [pltpu.VMEM((GCH * TRANS // 128, 128), jnp.float32)
              for _ in range(NB)]),
            [pltpu.SemaphoreType.DMA for _ in range(NB)],
            [pltpu.SemaphoreType.DMA for _ in range(NB)],
            pltpu.VMEM_SHARED((N_PAD, TRANS), jnp.float32),
            pltpu.VMEM_SHARED((N_PAD, TRANS), jnp.float32),
        ],
        compiler_params=_SC_PARAMS,
    )
    return kfn(ps, pd, sp1, dm1)


def _mlp_body(g_ref, asp_ref, adm_ref, w1t_ref, w1asp_ref, w1adm_ref,
              b1_ref, w2_ref, b2_ref, o_ref):
    for gi in range(NUM_CORES):
        trans = jnp.tanh(g_ref[gi])
        a_ref = asp_ref if gi == 0 else adm_ref
        w1a_ref = w1asp_ref if gi == 0 else w1adm_ref
        h = (jnp.dot(trans, w1t_ref[gi],
                     preferred_element_type=jnp.float32)
             + jnp.dot(a_ref[...], w1a_ref[...],
                       preferred_element_type=jnp.float32)
             + b1_ref[gi])
        h = jnp.maximum(h, 0.0)
        ew = jnp.dot(h, w2_ref[gi], preferred_element_type=jnp.float32)
        o_ref[gi] = jax.nn.sigmoid(ew + b2_ref[gi])


def _edge_mlp(g2, asp2, adm2, w1t_big, w1asp_big, w1adm_big,
              b1_big, w2_big, b2_big):
    grid = (E // BE,)
    bp = BE // PACK
    full = lambda a: pl.BlockSpec(a.shape, lambda bi: (0,) * a.ndim)
    return pl.pallas_call(
        _mlp_body,
        grid=grid,
        in_specs=[
            pl.BlockSpec((NUM_CORES, bp, PACK * TRANS), lambda bi: (0, bi, 0)),
            pl.BlockSpec((bp, PACK * 4), lambda bi: (bi, 0)),
            pl.BlockSpec((bp, PACK * 1), lambda bi: (bi, 0)),
            full(w1t_big),
            full(w1asp_big),
            full(w1adm_big),
            full(b1_big),
            full(w2_big),
            full(b2_big),
        ],
        out_specs=pl.BlockSpec((NUM_CORES, bp, PACK), lambda bi: (0, bi, 0)),
        out_shape=jax.ShapeDtypeStruct((NUM_CORES, E // PACK, PACK),
                                       jnp.float32),
    )(g2, asp2, adm2, w1t_big, w1asp_big, w1adm_big, b1_big, w2_big, b2_big)


def _prop_body(sp_hbm, dm_hbm, ew_hbm, m0_hbm, out_hbm,
               src_t, dst_t, ew_t, m_cur, acc,
               shared_s, shared_m, rsem):
        c = lax.axis_index("c")
        t = lax.axis_index("s")
        ebase = t * EPT
        sbase = t * SLICE

        @pl.when(c == 0)
        def _():
            pltpu.sync_copy(sp_hbm.at[pl.ds(ebase, EPT)], src_t)
            pltpu.sync_copy(sp_hbm.at[pl.ds(E + ebase, EPT)], dst_t)

        @pl.when(c == 1)
        def _():
            pltpu.sync_copy(dm_hbm.at[pl.ds(ebase, EPT)], src_t)
            pltpu.sync_copy(dm_hbm.at[pl.ds(E + ebase, EPT)], dst_t)

        pltpu.sync_copy(ew_hbm.at[c, pl.ds(ebase // PACK, EPT // PACK)], ew_t)
        pltpu.sync_copy(m0_hbm, m_cur)
        pltpu.sync_copy(m0_hbm, acc)
        lane = lax.iota(jnp.int32, LANES)
        qoff = lane // PACK
        jidx = lane % PACK

        def scatter_chunk(j):
            off = j * LANES
            si = src_t[pl.ds(off, LANES)]
            di = dst_t[pl.ds(off, LANES)]
            w = plsc.load_gather(ew_t, [2 * j + qoff, jidx])
            ms = plsc.load_gather(m_cur, [si])
            kk, vv = plsc.sort_key_val(di, ms * w)
            for sh in (1, 2, 4, 8):
                idx2 = jnp.maximum(lane - sh, 0)
                kp = kk.at[idx2].get(mode="promise_in_bounds")
                vp = vv.at[idx2].get(mode="promise_in_bounds")
                hit = (lane >= sh) & (kp == kk)
                vv = jnp.where(hit, jnp.maximum(vv, vp), vv)
            nxt = jnp.minimum(lane + 1, LANES - 1)
            kn = kk.at[nxt].get(mode="promise_in_bounds")
            is_last = (lane == LANES - 1) | (kn != kk)
            cur = plsc.load_gather(acc, [kk], mask=is_last)
            plsc.store_scatter(acc, [kk], jnp.maximum(vv, cur), mask=is_last)

        def edge_body(ju, _):
            scatter_chunk(ju)
            return 0

        for k in range(K):
            lax.fori_loop(0, EPT // LANES, edge_body, 0)
            pltpu.sync_copy(acc, shared_s.at[t])
            plsc.subcore_barrier()
            stage_in = [
                pltpu.async_copy(shared_s.at[jj, pl.ds(sbase, SLICE)],
                                 m_cur.at[pl.ds(jj * SLICE, SLICE)], rsem)
                for jj in range(NUM_SUBCORES)]
            for d in stage_in:
                d.wait()

            def red_body(r, _):
                o = r * LANES
                v = m_cur[pl.ds(o, LANES)]
                for jj in range(1, NUM_SUBCORES):
                    v = jnp.maximum(v, m_cur[pl.ds(jj * SLICE + o, LANES)])
                acc[pl.ds(o, LANES)] = v
                return 0

            lax.fori_loop(0, SLICE // LANES, red_body, 0)
            if k + 1 < K:
                pltpu.sync_copy(acc.at[pl.ds(0, SLICE)],
                                shared_m.at[pl.ds(sbase, SLICE)])
                plsc.subcore_barrier()
                pltpu.sync_copy(shared_m, m_cur)
                pltpu.sync_copy(shared_m, acc)
            else:
                pltpu.sync_copy(acc.at[pl.ds(0, SLICE)],
                                out_hbm.at[c, pl.ds(sbase, SLICE)])


def _propagate(sp1, dm1, ew3, m0):
    kfn = pl.kernel(
        _prop_body,
        out_type=jax.ShapeDtypeStruct((NUM_CORES, N_PAD), jnp.float32),
        mesh=_mesh(),
        scratch_types=[
            pltpu.VMEM((EPT,), jnp.int32),
            pltpu.VMEM((EPT,), jnp.int32),
            pltpu.VMEM((EPT // PACK, PACK), jnp.float32),
            pltpu.VMEM((N_PAD,), jnp.float32),
            pltpu.VMEM((N_PAD,), jnp.float32),
            pltpu.VMEM_SHARED((NUM_SUBCORES, N_PAD), jnp.float32),
            pltpu.VMEM_SHARED((N_PAD,), jnp.float32),
            pltpu.SemaphoreType.DMA,
        ],
        compiler_params=_SC_PARAMS,
    )
    return kfn(sp1, dm1, ew3, m0)


def _combine_body(prop_ref, mask_ref, o_ref):
    o_ref[...] = jnp.maximum(
        jnp.maximum(prop_ref[0:1, :], prop_ref[1:2, :]), mask_ref[...])


def _combine(prop, mask_pad2):
    return pl.pallas_call(
        _combine_body,
        out_shape=jax.ShapeDtypeStruct((1, N_PAD), jnp.float32),
    )(prop, mask_pad2)


def kernel(x, spatial_edge_index, dom_edge_index, spatial_edge_attr,
           dom_edge_attr, mask, W_ea, b_ea, W_d1, b_d1, W_d2, b_d2,
           W_p1, b_p1, W_p2, b_p2):
    f32 = jnp.float32

    x_pad = jnp.pad(x.astype(f32), ((0, N_PAD - N), (0, 0)))
    w_s = W_ea[:HIDDEN]
    w_d = W_ea[HIDDEN:]
    b_ea2 = b_ea.reshape(1, TRANS)

    sp1 = spatial_edge_index.astype(jnp.int32).reshape(2 * E)
    dm1 = dom_edge_index.astype(jnp.int32).reshape(2 * E)

    asp2 = (spatial_edge_attr.astype(f32).T.reshape(4, E // PACK, PACK)
            .transpose(1, 2, 0).reshape(E // PACK, PACK * 4))
    adm2 = (dom_edge_attr.astype(f32).T.reshape(1, E // PACK, PACK)
            .transpose(1, 2, 0).reshape(E // PACK, PACK * 1))

    w1t = jnp.stack([W_p1[4:], W_d1[1:]])
    b1 = jnp.stack([b_p1, b_d1])
    w2 = jnp.stack([W_p2, W_d2])
    b2 = jnp.stack([b_p2, b_d2])
    eye8 = jnp.eye(PACK, dtype=f32)

    def blkdiag3(w):
        return (eye8[None, :, None, :, None]
                * w[:, None, :, None, :]
                ).reshape(w.shape[0], PACK * w.shape[1], PACK * w.shape[2])

    w1t_big = blkdiag3(w1t)
    w1asp_big = blkdiag3(W_p1[None, :4])[0]
    w1adm_big = blkdiag3(W_d1[None, :1])[0]
    b1_big = jnp.tile(b1, (1, PACK))
    w2_big = blkdiag3(w2)
    b2_big = jnp.tile(b2, (1, PACK))

    m0 = jnp.pad(mask[:, 0].astype(f32), (0, N_PAD - N))
    mask_pad2 = m0.reshape(1, N_PAD)

    ps, pd = _node_proj(x_pad, w_s, w_d, b_ea2)
    g2 = _edge_gather(ps, pd, sp1, dm1)
    ew3 = _edge_mlp(g2, asp2, adm2, w1t_big, w1asp_big, w1adm_big,
                    b1_big, w2_big, b2_big)
    prop = _propagate(sp1, dm1, ew3, m0)
    out = _combine(prop, mask_pad2)
    return out[0, :N].reshape(N, 1)
